# baseline (speedup 1.0000x reference)
import functools

import jax
import jax.numpy as jnp
from jax.experimental import pallas as pl
from jax.experimental.pallas import tpu as pltpu

N = 16384
C = 1000
K = N // 2
BLOCK_R = 2048
NUM_B = N // BLOCK_R


def _losses_body(x, lbl):
    m = jnp.max(x, axis=1, keepdims=True)
    se = jnp.sum(jnp.exp(x - m), axis=1)
    cols = jax.lax.broadcasted_iota(jnp.int32, (BLOCK_R, C), 1)
    xlab = jnp.sum(jnp.where(cols == lbl[:, None], x, 0.0), axis=1)
    return jnp.log(se) + m[:, 0] - xlab


def _losses_kernel(x0_ref, x1_ref, lab_ref, out0_ref, out1_ref):
    lbl = lab_ref[0, 0, :]
    out0_ref[0, 0, :] = _losses_body(x0_ref[...], lbl[:BLOCK_R])
    out1_ref[0, 0, :] = _losses_body(x1_ref[...], lbl[BLOCK_R:])


def _losses(logits, labels32):
    half = NUM_B // 2
    labs = labels32.reshape(2, half, BLOCK_R)
    labels3 = jnp.concatenate(
        [labs[0], labs[1]], axis=-1).reshape(half, 1, 2 * BLOCK_R)
    out = pl.pallas_call(
        _losses_kernel,
        grid=(half,),
        in_specs=[
            pl.BlockSpec((BLOCK_R, C), lambda i: (i, 0)),
            pl.BlockSpec((BLOCK_R, C), lambda i, h=half: (i + h, 0)),
            pl.BlockSpec((1, 1, 2 * BLOCK_R), lambda i: (i, 0, 0)),
        ],
        out_specs=[
            pl.BlockSpec((1, 1, BLOCK_R), lambda i: (i, 0, 0)),
            pl.BlockSpec((1, 1, BLOCK_R), lambda i: (i, 0, 0)),
        ],
        out_shape=[
            jax.ShapeDtypeStruct((half, 1, BLOCK_R), jnp.float32),
            jax.ShapeDtypeStruct((half, 1, BLOCK_R), jnp.float32),
        ],
    )(logits, logits, labels3)
    return jnp.concatenate([o.reshape(-1) for o in out])


def _finalize_kernel(l_ref, out_ref):
    vals = l_ref[...]
    bits = jax.lax.bitcast_convert_type(vals, jnp.int32)

    def body(_, carry):
        lo, hi = carry
        mid = lo + (hi - lo) // 2
        cnt = jnp.sum((bits >= mid).astype(jnp.int32))
        take = cnt >= K
        return jnp.where(take, mid, lo), jnp.where(take, hi, mid)

    t_int, _ = jax.lax.fori_loop(
        0, 31, body, (jnp.int32(0), jnp.int32(0x7F800000)))
    t = jax.lax.bitcast_convert_type(t_int, jnp.float32)
    gt = bits > t_int
    cnt_gt = jnp.sum(gt.astype(jnp.int32))
    sum_gt = jnp.sum(jnp.where(gt, vals, 0.0))
    out_ref[0, 0] = (sum_gt + (K - cnt_gt).astype(jnp.float32) * t) / K


def _finalize(losses):
    return pl.pallas_call(
        _finalize_kernel,
        out_specs=pl.BlockSpec(memory_space=pltpu.SMEM),
        out_shape=jax.ShapeDtypeStruct((1, 1), jnp.float32),
    )(losses.reshape(N // 128, 128))


@jax.jit
def kernel(logits, labels):
    labels32 = labels.astype(jnp.int32)
    losses = _losses(logits, labels32)
    return _finalize(losses)[0, 0]
